# baseline (speedup 1.0000x reference)
import functools

import jax
import jax.numpy as jnp
from jax import lax
from jax.experimental import pallas as pl
from jax.experimental.pallas import tpu as pltpu
from jax.experimental.pallas import tpu_sc as plsc

_N = 10000
_E = 320000
_EH = _E // 2
_H = 128
_L = 4

_NC = 2
_NS = 16
_NW = _NC * _NS
_EW = 4992
_TAILN = _EH - _NW * _EW
_CT = 64
_NTAIL = _TAILN // _CT

_CG = 312
_NGB = 2
_NGJ = _EW // _CG // _NGB

_CS = 96
_NSB = 2
_NSJ = _EW // _CS // _NSB


def _sc_mesh():
    return plsc.VectorSubcoreMesh(
        core_axis_name="c", subcore_axis_name="s",
        num_cores=_NC, num_subcores=_NS)


_SC_PARAMS = pltpu.CompilerParams(use_tc_tiling_on_sc=False)


def _make_gather(width, base0):
    @functools.partial(
        pl.kernel,
        out_type=jax.ShapeDtypeStruct((_EH, width), jnp.float32),
        mesh=_sc_mesh(),
        compiler_params=_SC_PARAMS,
        scratch_types=[
            pltpu.VMEM((_EW,), jnp.int32),
            pltpu.VMEM((_CT,), jnp.int32),
            [pltpu.VMEM((_CG, width), jnp.float32)] * _NGB,
            [pltpu.SemaphoreType.DMA] * _NGB,
            [pltpu.SemaphoreType.DMA] * _NGB,
        ],
    )
    def gather(tbl, idx_hbm, out, idx_all, idx_t, bufs, gsems, ssems):
        w = lax.axis_index("s") * _NC + lax.axis_index("c")
        base = pl.multiple_of(w * _EW, 8)
        pltpu.sync_copy(idx_hbm.at[pl.ds(base0 + base, _EW)], idx_all)

        def ring(j, carry):
            descs = []
            for t in range(_NGB):
                off = pl.multiple_of((j * _NGB + t) * _CG, 8)

                @pl.when(j > 0)
                def _drain_store(t=t):
                    pltpu.make_async_copy(
                        out.at[pl.ds(base, _CG)], bufs[t], ssems[t]).wait()

                descs.append(pltpu.async_copy(
                    tbl.at[idx_all.at[pl.ds(off, _CG)]], bufs[t], gsems[t]))
            for t in range(_NGB):
                off = pl.multiple_of((j * _NGB + t) * _CG, 8)
                descs[t].wait()
                pltpu.async_copy(bufs[t], out.at[pl.ds(base + off, _CG)],
                                 ssems[t])
            return carry

        lax.fori_loop(0, _NGJ, ring, 0)
        for t in range(_NGB):
            pltpu.make_async_copy(
                out.at[pl.ds(base, _CG)], bufs[t], ssems[t]).wait()

        @pl.when(w < _NTAIL)
        def _tail():
            tb = pl.multiple_of(_NW * _EW + w * _CT, 8)
            pltpu.sync_copy(idx_hbm.at[pl.ds(base0 + tb, _CT)], idx_t)
            pltpu.async_copy(tbl.at[idx_t], bufs[0].at[pl.ds(0, _CT)],
                             gsems[0]).wait()
            pltpu.sync_copy(bufs[0].at[pl.ds(0, _CT)],
                            out.at[pl.ds(tb, _CT)])

    return gather


def _make_scatter(width, nzrow, base0):
    nz = _N // nzrow

    @functools.partial(
        pl.kernel,
        out_type=jax.ShapeDtypeStruct((_NC * _N, width), jnp.float32),
        mesh=_sc_mesh(),
        compiler_params=_SC_PARAMS,
        scratch_types=[
            pltpu.VMEM_SHARED((_N, width), jnp.float32),
            pltpu.VMEM((_EW,), jnp.int32),
            pltpu.VMEM((_CT,), jnp.int32),
            [pltpu.VMEM((_CS, width), jnp.float32)] * _NSB,
            [pltpu.SemaphoreType.DMA] * _NSB,
            [pltpu.SemaphoreType.DMA] * _NSB,
        ],
    )
    def scatter(msg, idx_hbm, zeros_hbm, out,
                acc, idx_all, idx_t, bufs, lsems, asems):
        cid = lax.axis_index("c")
        sid = lax.axis_index("s")
        w = sid * _NC + cid

        @pl.when(sid < nz)
        def _zero():
            pltpu.sync_copy(zeros_hbm, acc.at[pl.ds(sid * nzrow, nzrow)])

        plsc.subcore_barrier()

        base = pl.multiple_of(w * _EW, 8)
        pltpu.sync_copy(idx_hbm.at[pl.ds(base0 + base, _EW)], idx_all)

        def ring(j, carry):
            descs = []
            for t in range(_NSB):
                off = pl.multiple_of((j * _NSB + t) * _CS, 8)

                @pl.when(j > 0)
                def _drain_add(t=t):
                    pltpu.make_async_copy(
                        msg.at[pl.ds(base, _CS)], bufs[t], asems[t]).wait()

                descs.append(pltpu.async_copy(
                    msg.at[pl.ds(base + off, _CS)], bufs[t], lsems[t]))
            for t in range(_NSB):
                off = pl.multiple_of((j * _NSB + t) * _CS, 8)
                descs[t].wait()
                pltpu.async_copy(bufs[t], acc.at[idx_all.at[pl.ds(off, _CS)]],
                                 asems[t], add=True)
            return carry

        lax.fori_loop(0, _NSJ, ring, 0)
        for t in range(_NSB):
            pltpu.make_async_copy(
                msg.at[pl.ds(base, _CS)], bufs[t], asems[t]).wait()

        @pl.when(w < _NTAIL)
        def _tail():
            tb = pl.multiple_of(_NW * _EW + w * _CT, 8)
            pltpu.sync_copy(idx_hbm.at[pl.ds(base0 + tb, _CT)], idx_t)
            pltpu.sync_copy(msg.at[pl.ds(tb, _CT)], bufs[0].at[pl.ds(0, _CT)])
            pltpu.sync_copy(bufs[0].at[pl.ds(0, _CT)], acc.at[idx_t],
                            add=True)

        plsc.subcore_barrier()

        @pl.when(sid < nz)
        def _writeback():
            pltpu.sync_copy(acc.at[pl.ds(sid * nzrow, nzrow)],
                            out.at[pl.ds(cid * _N + sid * nzrow, nzrow)])

    return scatter


def _make_scatter2(base0):
    width = 8
    nzrow = 1250
    nz = _N // nzrow

    @functools.partial(
        pl.kernel,
        out_type=jax.ShapeDtypeStruct((_NC * _N, width), jnp.float32),
        mesh=_sc_mesh(),
        compiler_params=_SC_PARAMS,
        scratch_types=[
            pltpu.VMEM_SHARED((_N, width), jnp.float32),
            pltpu.VMEM((_EW,), jnp.int32),
            pltpu.VMEM((_CT,), jnp.int32),
            [pltpu.VMEM((_CS, width), jnp.float32)] * _NSB,
            [pltpu.SemaphoreType.DMA] * _NSB,
            [pltpu.SemaphoreType.DMA] * _NSB,
        ],
    )
    def scatter2(pf, nf, row_hbm, col_hbm, zeros_hbm, out,
                 acc, idx_all, idx_t, bufs, lsems, asems):
        cid = lax.axis_index("c")
        sid = lax.axis_index("s")
        w = sid * _NC + cid

        @pl.when(sid < nz)
        def _zero():
            pltpu.sync_copy(zeros_hbm, acc.at[pl.ds(sid * nzrow, nzrow)])

        plsc.subcore_barrier()

        base = pl.multiple_of(w * _EW, 8)

        def phase(vals, idx_hbm):
            pltpu.sync_copy(idx_hbm.at[pl.ds(base0 + base, _EW)], idx_all)

            def ring(j, carry):
                descs = []
                for t in range(_NSB):
                    off = pl.multiple_of((j * _NSB + t) * _CS, 8)

                    @pl.when(j > 0)
                    def _drain_add(t=t):
                        pltpu.make_async_copy(
                            vals.at[pl.ds(base, _CS)], bufs[t],
                            asems[t]).wait()

                    descs.append(pltpu.async_copy(
                        vals.at[pl.ds(base + off, _CS)], bufs[t], lsems[t]))
                for t in range(_NSB):
                    off = pl.multiple_of((j * _NSB + t) * _CS, 8)
                    descs[t].wait()
                    pltpu.async_copy(
                        bufs[t], acc.at[idx_all.at[pl.ds(off, _CS)]],
                        asems[t], add=True)
                return carry

            lax.fori_loop(0, _NSJ, ring, 0)
            for t in range(_NSB):
                pltpu.make_async_copy(
                    vals.at[pl.ds(base, _CS)], bufs[t], asems[t]).wait()

            @pl.when(w < _NTAIL)
            def _tail():
                tb = pl.multiple_of(_NW * _EW + w * _CT, 8)
                pltpu.sync_copy(idx_hbm.at[pl.ds(base0 + tb, _CT)], idx_t)
                pltpu.sync_copy(vals.at[pl.ds(tb, _CT)],
                                bufs[0].at[pl.ds(0, _CT)])
                pltpu.sync_copy(bufs[0].at[pl.ds(0, _CT)], acc.at[idx_t],
                                add=True)

        phase(pf, row_hbm)
        phase(nf, col_hbm)

        plsc.subcore_barrier()

        @pl.when(sid < nz)
        def _writeback():
            pltpu.sync_copy(acc.at[pl.ds(sid * nzrow, nzrow)],
                            out.at[pl.ds(cid * _N + sid * nzrow, nzrow)])

    return scatter2


_gather8_h = (_make_gather(8, 0), _make_gather(8, _EH))
_gather128_h = (_make_gather(_H, 0), _make_gather(_H, _EH))
_gather136_h = (_make_gather(_H + 8, 0), _make_gather(_H + 8, _EH))
_scatter128_h = (_make_scatter(_H, _N // _NS, 0),
                 _make_scatter(_H, _N // _NS, _EH))
_scatter2_h = (_make_scatter2(0), _make_scatter2(_EH))


_BN = 1000
_GN = _N // _BN
_BE = 2000
_GE = _EH // _BE

_PAR = pltpu.CompilerParams(dimension_semantics=("parallel",))


def _silu(x):
    return x * jax.nn.sigmoid(x)


def _full(shape):
    return pl.BlockSpec(shape, lambda i: (0, 0))


def _g0_body(vel_ref, w_ref, b_ref, pos8_ref, out_ref):
    v = vel_ref[...]
    s = jnp.sum(w_ref[...], axis=0, keepdims=True)
    vn = jnp.sqrt(jnp.sum(v * v, axis=1, keepdims=True))
    out_ref[...] = jnp.concatenate([vn * s + b_ref[...], pos8_ref[...]],
                                   axis=1)


_g0_call = pl.pallas_call(
    _g0_body,
    grid=(_GN,),
    in_specs=[pl.BlockSpec((_BN, 3), lambda i: (i, 0)),
              _full((_H, _H)), _full((1, _H)),
              pl.BlockSpec((_BN, 8), lambda i: (i, 0))],
    out_specs=pl.BlockSpec((_BN, _H + 8), lambda i: (i, 0)),
    out_shape=jax.ShapeDtypeStruct((_N, _H + 8), jnp.float32),
    compiler_params=_PAR,
)


def _g_body(a_ref, b_ref, c_ref, d_ref, w_ref, bias_ref, out_ref):
    nf = a_ref[...] + b_ref[...] + c_ref[...] + d_ref[...]
    out_ref[...] = (jnp.dot(nf, w_ref[...], preferred_element_type=jnp.float32)
                    + bias_ref[...])


_g_call = pl.pallas_call(
    _g_body,
    grid=(_GN,),
    in_specs=[pl.BlockSpec((_BN, _H), lambda i: (i, 0)),
              pl.BlockSpec((_BN, _H), lambda i: (i + _GN, 0)),
              pl.BlockSpec((_BN, _H), lambda i: (i, 0)),
              pl.BlockSpec((_BN, _H), lambda i: (i + _GN, 0)),
              _full((_H, _H)), _full((1, _H))],
    out_specs=pl.BlockSpec((_BN, _H), lambda i: (i, 0)),
    out_shape=jax.ShapeDtypeStruct((_N, _H), jnp.float32),
    compiler_params=_PAR,
)


def _edge_attr(pr, pc):
    rd = pr - pc
    d = jnp.sqrt(jnp.sum(rd * rd, axis=1, keepdims=True))
    return rd, d


def _am0_body(gp_ref, posc_ref, eew1_ref, eeb1_ref, eew2_ref,
              eeb2_ref, w1t_ref, w2_ref, b2_ref, emb_ref, msg_ref, dir_ref):
    gp = gp_ref[...]
    g = gp[:, :_H]
    pr = gp[:, _H:]
    rd, d = _edge_attr(pr, posc_ref[...])
    dir_ref[...] = rd / (d + 1e-8)
    lane = lax.broadcasted_iota(jnp.int32, rd.shape, 1)
    ea = jnp.where(lane < 3, rd, jnp.where(lane == 3, d, 0.0))
    h = _silu(jnp.dot(ea, eew1_ref[...], preferred_element_type=jnp.float32)
              + eeb1_ref[...])
    emb = (jnp.dot(h, eew2_ref[...], preferred_element_type=jnp.float32)
           + eeb2_ref[...])
    emb_ref[...] = emb.astype(jnp.bfloat16)
    h2 = _silu(jnp.dot(emb, w1t_ref[...], preferred_element_type=jnp.float32)
               + g)
    msg_ref[...] = (jnp.dot(h2, w2_ref[...], preferred_element_type=jnp.float32)
                    + b2_ref[...])


_am0_call = pl.pallas_call(
    _am0_body,
    grid=(_GE,),
    in_specs=[pl.BlockSpec((_BE, _H + 8), lambda i: (i, 0)),
              pl.BlockSpec((_BE, 8), lambda i: (i, 0)),
              _full((8, _H)), _full((1, _H)), _full((_H, _H)), _full((1, _H)),
              _full((_H, _H)), _full((_H, _H)), _full((1, _H))],
    out_specs=[pl.BlockSpec((_BE, _H), lambda i: (i, 0)),
               pl.BlockSpec((_BE, _H), lambda i: (i, 0)),
               pl.BlockSpec((_BE, 8), lambda i: (i, 0))],
    out_shape=[jax.ShapeDtypeStruct((_EH, _H), jnp.bfloat16),
               jax.ShapeDtypeStruct((_EH, _H), jnp.float32),
               jax.ShapeDtypeStruct((_EH, 8), jnp.float32)],
    compiler_params=_PAR,
)


def _msg_body(emb_ref, g_ref, w1t_ref, w2_ref, b2_ref, msg_ref):
    h = _silu(jnp.dot(emb_ref[...].astype(jnp.float32), w1t_ref[...],
                      preferred_element_type=jnp.float32) + g_ref[...])
    msg_ref[...] = (jnp.dot(h, w2_ref[...], preferred_element_type=jnp.float32)
                    + b2_ref[...])


_msg_call = pl.pallas_call(
    _msg_body,
    grid=(_GE,),
    in_specs=[pl.BlockSpec((_BE, _H), lambda i: (i, 0)),
              pl.BlockSpec((_BE, _H), lambda i: (i, 0)),
              _full((_H, _H)), _full((_H, _H)), _full((1, _H))],
    out_specs=pl.BlockSpec((_BE, _H), lambda i: (i, 0)),
    out_shape=jax.ShapeDtypeStruct((_EH, _H), jnp.float32),
    compiler_params=_PAR,
)


def _ff_body(g_ref, w2_ref, b2_ref, dir_ref, pf_ref, nf_ref):
    fm8 = (jnp.dot(_silu(g_ref[...]), w2_ref[...],
                   preferred_element_type=jnp.float32) + b2_ref[...])
    fm = fm8[:, 0:1]
    pf = fm * dir_ref[...]
    pf_ref[...] = pf
    nf_ref[...] = -pf


_ff_call = pl.pallas_call(
    _ff_body,
    grid=(_GE,),
    in_specs=[pl.BlockSpec((_BE, _H), lambda i: (i, 0)),
              _full((_H, 8)), _full((1, 8)),
              pl.BlockSpec((_BE, 8), lambda i: (i, 0))],
    out_specs=[pl.BlockSpec((_BE, 8), lambda i: (i, 0)),
               pl.BlockSpec((_BE, 8), lambda i: (i, 0))],
    out_shape=[jax.ShapeDtypeStruct((_EH, 8), jnp.float32),
               jax.ShapeDtypeStruct((_EH, 8), jnp.float32)],
    compiler_params=_PAR,
)


def _fin_body(a_ref, b_ref, c_ref, d_ref, out_ref):
    out_ref[...] = (a_ref[...] + b_ref[...] + c_ref[...] + d_ref[...])[:, :3]


_fin_call = pl.pallas_call(
    _fin_body,
    grid=(_GN,),
    in_specs=[pl.BlockSpec((_BN, 8), lambda i: (i, 0)),
              pl.BlockSpec((_BN, 8), lambda i: (i + _GN, 0)),
              pl.BlockSpec((_BN, 8), lambda i: (i, 0)),
              pl.BlockSpec((_BN, 8), lambda i: (i + _GN, 0))],
    out_specs=pl.BlockSpec((_BN, 3), lambda i: (i, 0)),
    out_shape=jax.ShapeDtypeStruct((_N, 3), jnp.float32),
    compiler_params=_PAR,
)


def kernel(pos, vel, masses, edge_index, ee_w1, ee_b1, ee_w2, ee_b2,
           msg_w1, msg_b1, msg_w2, msg_b2, fd_w1, fd_b1, fd_w2, fd_b2):
    f32 = jnp.float32
    row = edge_index[0]
    col = edge_index[1]
    pos8 = jnp.concatenate([pos, jnp.zeros((_N, 5), f32)], axis=1)
    eew1p = jnp.concatenate([ee_w1, jnp.zeros((4, _H), f32)], axis=0)
    fd_w2p = jnp.concatenate([fd_w2, jnp.zeros((_H, 7), f32)], axis=1)
    fd_b2p = jnp.concatenate([fd_b2, jnp.zeros((7,), f32)]).reshape(1, 8)
    w1t = msg_w1[:, :_H, :]
    w1b = msg_w1[:, _H:, :]
    zeros128 = jnp.zeros((_N // _NS, _H), f32)
    zeros8 = jnp.zeros((1250, 8), f32)

    g0 = _g0_call(vel, w1b[0], msg_b1[0].reshape(1, _H), pos8)
    posc = [_gather8_h[h](pos8, col) for h in range(2)]
    gth = [_gather136_h[h](g0, row) for h in range(2)]
    emb, msg, p = [None, None], [None, None], [None, None]
    dir8 = [None, None]
    for h in range(2):
        emb[h], msg[h], dir8[h] = _am0_call(
            gth[h], posc[h], eew1p, ee_b1.reshape(1, _H),
            ee_w2, ee_b2.reshape(1, _H), w1t[0], msg_w2[0],
            msg_b2[0].reshape(1, _H))
        p[h] = _scatter128_h[h](msg[h], col, zeros128)
    for l in range(1, _L):
        g = _g_call(p[0], p[0], p[1], p[1], w1b[l], msg_b1[l].reshape(1, _H))
        for h in range(2):
            gth[h] = _gather128_h[h](g, row)
        for h in range(2):
            msg[h] = _msg_call(emb[h], gth[h], w1t[l], msg_w2[l],
                               msg_b2[l].reshape(1, _H))
            p[h] = _scatter128_h[h](msg[h], col, zeros128)
    gf = _g_call(p[0], p[0], p[1], p[1], fd_w1, fd_b1.reshape(1, _H))
    q = [None, None]
    for h in range(2):
        gfr = _gather128_h[h](gf, row)
        pf, nf = _ff_call(gfr, fd_w2p, fd_b2p, dir8[h])
        q[h] = _scatter2_h[h](pf, nf, row, col, zeros8)
    return _fin_call(q[0], q[0], q[1], q[1])


# speedup vs baseline: 1.1171x; 1.1171x over previous
import functools

import jax
import jax.numpy as jnp
from jax import lax
from jax.experimental import pallas as pl
from jax.experimental.pallas import tpu as pltpu
from jax.experimental.pallas import tpu_sc as plsc

_N = 10000
_E = 320000
_EH = _E // 2
_H = 128
_L = 4

_NC = 2
_NS = 16
_NW = _NC * _NS
_EW = 4992
_TAILN = _EH - _NW * _EW
_CT = 64
_NTAIL = _TAILN // _CT

_CG = 312
_NGB = 2
_NGJ = _EW // _CG // _NGB

_CS = 96
_NSB = 2
_NSJ = _EW // _CS // _NSB


def _sc_mesh():
    return plsc.VectorSubcoreMesh(
        core_axis_name="c", subcore_axis_name="s",
        num_cores=_NC, num_subcores=_NS)


_SC_PARAMS = pltpu.CompilerParams(use_tc_tiling_on_sc=False)


def _make_gather(width, base0):
    @functools.partial(
        pl.kernel,
        out_type=jax.ShapeDtypeStruct((_EH, width), jnp.float32),
        mesh=_sc_mesh(),
        compiler_params=_SC_PARAMS,
        scratch_types=[
            pltpu.VMEM((_EW,), jnp.int32),
            pltpu.VMEM((_CT,), jnp.int32),
            [pltpu.VMEM((_CG, width), jnp.float32)] * _NGB,
            [pltpu.SemaphoreType.DMA] * _NGB,
            [pltpu.SemaphoreType.DMA] * _NGB,
        ],
    )
    def gather(tbl, idx_hbm, out, idx_all, idx_t, bufs, gsems, ssems):
        w = lax.axis_index("s") * _NC + lax.axis_index("c")
        base = pl.multiple_of(w * _EW, 8)
        pltpu.sync_copy(idx_hbm.at[pl.ds(base0 + base, _EW)], idx_all)

        def ring(j, carry):
            descs = []
            for t in range(_NGB):
                off = pl.multiple_of((j * _NGB + t) * _CG, 8)

                @pl.when(j > 0)
                def _drain_store(t=t):
                    pltpu.make_async_copy(
                        out.at[pl.ds(base, _CG)], bufs[t], ssems[t]).wait()

                descs.append(pltpu.async_copy(
                    tbl.at[idx_all.at[pl.ds(off, _CG)]], bufs[t], gsems[t]))
            for t in range(_NGB):
                off = pl.multiple_of((j * _NGB + t) * _CG, 8)
                descs[t].wait()
                pltpu.async_copy(bufs[t], out.at[pl.ds(base + off, _CG)],
                                 ssems[t])
            return carry

        lax.fori_loop(0, _NGJ, ring, 0)
        for t in range(_NGB):
            pltpu.make_async_copy(
                out.at[pl.ds(base, _CG)], bufs[t], ssems[t]).wait()

        @pl.when(w < _NTAIL)
        def _tail():
            tb = pl.multiple_of(_NW * _EW + w * _CT, 8)
            pltpu.sync_copy(idx_hbm.at[pl.ds(base0 + tb, _CT)], idx_t)
            pltpu.async_copy(tbl.at[idx_t], bufs[0].at[pl.ds(0, _CT)],
                             gsems[0]).wait()
            pltpu.sync_copy(bufs[0].at[pl.ds(0, _CT)],
                            out.at[pl.ds(tb, _CT)])

    return gather


def _make_scatter(width, nzrow, base0):
    nz = _N // nzrow

    @functools.partial(
        pl.kernel,
        out_type=jax.ShapeDtypeStruct((_NC * _N, width), jnp.float32),
        mesh=_sc_mesh(),
        compiler_params=_SC_PARAMS,
        scratch_types=[
            pltpu.VMEM_SHARED((_N, width), jnp.float32),
            pltpu.VMEM((_EW,), jnp.int32),
            pltpu.VMEM((_CT,), jnp.int32),
            [pltpu.VMEM((_CS, width), jnp.float32)] * _NSB,
            [pltpu.SemaphoreType.DMA] * _NSB,
            [pltpu.SemaphoreType.DMA] * _NSB,
        ],
    )
    def scatter(msg, idx_hbm, zeros_hbm, out,
                acc, idx_all, idx_t, bufs, lsems, asems):
        cid = lax.axis_index("c")
        sid = lax.axis_index("s")
        w = sid * _NC + cid

        @pl.when(sid < nz)
        def _zero():
            pltpu.sync_copy(zeros_hbm, acc.at[pl.ds(sid * nzrow, nzrow)])

        plsc.subcore_barrier()

        base = pl.multiple_of(w * _EW, 8)
        pltpu.sync_copy(idx_hbm.at[pl.ds(base0 + base, _EW)], idx_all)

        def ring(j, carry):
            descs = []
            for t in range(_NSB):
                off = pl.multiple_of((j * _NSB + t) * _CS, 8)

                @pl.when(j > 0)
                def _drain_add(t=t):
                    pltpu.make_async_copy(
                        msg.at[pl.ds(base, _CS)], bufs[t], asems[t]).wait()

                descs.append(pltpu.async_copy(
                    msg.at[pl.ds(base + off, _CS)], bufs[t], lsems[t]))
            for t in range(_NSB):
                off = pl.multiple_of((j * _NSB + t) * _CS, 8)
                descs[t].wait()
                pltpu.async_copy(bufs[t], acc.at[idx_all.at[pl.ds(off, _CS)]],
                                 asems[t], add=True)
            return carry

        lax.fori_loop(0, _NSJ, ring, 0)
        for t in range(_NSB):
            pltpu.make_async_copy(
                msg.at[pl.ds(base, _CS)], bufs[t], asems[t]).wait()

        @pl.when(w < _NTAIL)
        def _tail():
            tb = pl.multiple_of(_NW * _EW + w * _CT, 8)
            pltpu.sync_copy(idx_hbm.at[pl.ds(base0 + tb, _CT)], idx_t)
            pltpu.sync_copy(msg.at[pl.ds(tb, _CT)], bufs[0].at[pl.ds(0, _CT)])
            pltpu.sync_copy(bufs[0].at[pl.ds(0, _CT)], acc.at[idx_t],
                            add=True)

        plsc.subcore_barrier()

        @pl.when(sid < nz)
        def _writeback():
            pltpu.sync_copy(acc.at[pl.ds(sid * nzrow, nzrow)],
                            out.at[pl.ds(cid * _N + sid * nzrow, nzrow)])

    return scatter


def _make_scatter2(base0):
    width = 8
    nzrow = 1250
    nz = _N // nzrow

    @functools.partial(
        pl.kernel,
        out_type=jax.ShapeDtypeStruct((_NC * _N, width), jnp.float32),
        mesh=_sc_mesh(),
        compiler_params=_SC_PARAMS,
        scratch_types=[
            pltpu.VMEM_SHARED((_N, width), jnp.float32),
            pltpu.VMEM((_EW,), jnp.int32),
            pltpu.VMEM((_CT,), jnp.int32),
            [pltpu.VMEM((_CS, width), jnp.float32)] * _NSB,
            [pltpu.SemaphoreType.DMA] * _NSB,
            [pltpu.SemaphoreType.DMA] * _NSB,
        ],
    )
    def scatter2(pf, nf, row_hbm, col_hbm, zeros_hbm, out,
                 acc, idx_all, idx_t, bufs, lsems, asems):
        cid = lax.axis_index("c")
        sid = lax.axis_index("s")
        w = sid * _NC + cid

        @pl.when(sid < nz)
        def _zero():
            pltpu.sync_copy(zeros_hbm, acc.at[pl.ds(sid * nzrow, nzrow)])

        plsc.subcore_barrier()

        base = pl.multiple_of(w * _EW, 8)

        def phase(vals, idx_hbm):
            pltpu.sync_copy(idx_hbm.at[pl.ds(base0 + base, _EW)], idx_all)

            def ring(j, carry):
                descs = []
                for t in range(_NSB):
                    off = pl.multiple_of((j * _NSB + t) * _CS, 8)

                    @pl.when(j > 0)
                    def _drain_add(t=t):
                        pltpu.make_async_copy(
                            vals.at[pl.ds(base, _CS)], bufs[t],
                            asems[t]).wait()

                    descs.append(pltpu.async_copy(
                        vals.at[pl.ds(base + off, _CS)], bufs[t], lsems[t]))
                for t in range(_NSB):
                    off = pl.multiple_of((j * _NSB + t) * _CS, 8)
                    descs[t].wait()
                    pltpu.async_copy(
                        bufs[t], acc.at[idx_all.at[pl.ds(off, _CS)]],
                        asems[t], add=True)
                return carry

            lax.fori_loop(0, _NSJ, ring, 0)
            for t in range(_NSB):
                pltpu.make_async_copy(
                    vals.at[pl.ds(base, _CS)], bufs[t], asems[t]).wait()

            @pl.when(w < _NTAIL)
            def _tail():
                tb = pl.multiple_of(_NW * _EW + w * _CT, 8)
                pltpu.sync_copy(idx_hbm.at[pl.ds(base0 + tb, _CT)], idx_t)
                pltpu.sync_copy(vals.at[pl.ds(tb, _CT)],
                                bufs[0].at[pl.ds(0, _CT)])
                pltpu.sync_copy(bufs[0].at[pl.ds(0, _CT)], acc.at[idx_t],
                                add=True)

        phase(pf, row_hbm)
        phase(nf, col_hbm)

        plsc.subcore_barrier()

        @pl.when(sid < nz)
        def _writeback():
            pltpu.sync_copy(acc.at[pl.ds(sid * nzrow, nzrow)],
                            out.at[pl.ds(cid * _N + sid * nzrow, nzrow)])

    return scatter2


_gather8_h = (_make_gather(8, 0), _make_gather(8, _EH))
_gather128_h = (_make_gather(_H, 0), _make_gather(_H, _EH))
_scatter128_h = (_make_scatter(_H, _N // _NS, 0),
                 _make_scatter(_H, _N // _NS, _EH))
_scatter2_h = (_make_scatter2(0), _make_scatter2(_EH))


_BN = 1000
_GN = _N // _BN
_BE = 2000
_GE = _EH // _BE

_PAR = pltpu.CompilerParams(dimension_semantics=("parallel",))


def _silu(x):
    return x * jax.nn.sigmoid(x)


def _full(shape):
    return pl.BlockSpec(shape, lambda i: (0, 0))


def _g0_body(vel_ref, w_ref, b_ref, out_ref):
    v = vel_ref[...]
    s = jnp.sum(w_ref[...], axis=0, keepdims=True)
    vn = jnp.sqrt(jnp.sum(v * v, axis=1, keepdims=True))
    out_ref[...] = vn * s + b_ref[...]


_g0_call = pl.pallas_call(
    _g0_body,
    grid=(_GN,),
    in_specs=[pl.BlockSpec((_BN, 3), lambda i: (i, 0)),
              _full((_H, _H)), _full((1, _H))],
    out_specs=pl.BlockSpec((_BN, _H), lambda i: (i, 0)),
    out_shape=jax.ShapeDtypeStruct((_N, _H), jnp.float32),
    compiler_params=_PAR,
)


def _g_body(a_ref, b_ref, c_ref, d_ref, w_ref, bias_ref, out_ref):
    nf = a_ref[...] + b_ref[...] + c_ref[...] + d_ref[...]
    out_ref[...] = (jnp.dot(nf, w_ref[...], preferred_element_type=jnp.float32)
                    + bias_ref[...])


_g_call = pl.pallas_call(
    _g_body,
    grid=(_GN,),
    in_specs=[pl.BlockSpec((_BN, _H), lambda i: (i, 0)),
              pl.BlockSpec((_BN, _H), lambda i: (i + _GN, 0)),
              pl.BlockSpec((_BN, _H), lambda i: (i, 0)),
              pl.BlockSpec((_BN, _H), lambda i: (i + _GN, 0)),
              _full((_H, _H)), _full((1, _H))],
    out_specs=pl.BlockSpec((_BN, _H), lambda i: (i, 0)),
    out_shape=jax.ShapeDtypeStruct((_N, _H), jnp.float32),
    compiler_params=_PAR,
)


def _edge_attr(pr, pc):
    rd = pr - pc
    d = jnp.sqrt(jnp.sum(rd * rd, axis=1, keepdims=True))
    return rd, d


def _am0_body(posr_ref, posc_ref, g_ref, eew1_ref, eeb1_ref, eew2_ref,
              eeb2_ref, w1t_ref, w2_ref, b2_ref, emb_ref, msg_ref):
    rd, d = _edge_attr(posr_ref[...], posc_ref[...])
    lane = lax.broadcasted_iota(jnp.int32, rd.shape, 1)
    ea = jnp.where(lane < 3, rd, jnp.where(lane == 3, d, 0.0))
    h = _silu(jnp.dot(ea, eew1_ref[...], preferred_element_type=jnp.float32)
              + eeb1_ref[...])
    emb = (jnp.dot(h, eew2_ref[...], preferred_element_type=jnp.float32)
           + eeb2_ref[...])
    emb_ref[...] = emb.astype(jnp.bfloat16)
    h2 = _silu(jnp.dot(emb, w1t_ref[...], preferred_element_type=jnp.float32)
               + g_ref[...])
    msg_ref[...] = (jnp.dot(h2, w2_ref[...], preferred_element_type=jnp.float32)
                    + b2_ref[...])


_am0_call = pl.pallas_call(
    _am0_body,
    grid=(_GE,),
    in_specs=[pl.BlockSpec((_BE, 8), lambda i: (i, 0)),
              pl.BlockSpec((_BE, 8), lambda i: (i, 0)),
              pl.BlockSpec((_BE, _H), lambda i: (i, 0)),
              _full((8, _H)), _full((1, _H)), _full((_H, _H)), _full((1, _H)),
              _full((_H, _H)), _full((_H, _H)), _full((1, _H))],
    out_specs=[pl.BlockSpec((_BE, _H), lambda i: (i, 0)),
               pl.BlockSpec((_BE, _H), lambda i: (i, 0))],
    out_shape=[jax.ShapeDtypeStruct((_EH, _H), jnp.bfloat16),
               jax.ShapeDtypeStruct((_EH, _H), jnp.float32)],
    compiler_params=_PAR,
)


def _msg_body(emb_ref, g_ref, w1t_ref, w2_ref, b2_ref, msg_ref):
    h = _silu(jnp.dot(emb_ref[...].astype(jnp.float32), w1t_ref[...],
                      preferred_element_type=jnp.float32) + g_ref[...])
    msg_ref[...] = (jnp.dot(h, w2_ref[...], preferred_element_type=jnp.float32)
                    + b2_ref[...])


_msg_call = pl.pallas_call(
    _msg_body,
    grid=(_GE,),
    in_specs=[pl.BlockSpec((_BE, _H), lambda i: (i, 0)),
              pl.BlockSpec((_BE, _H), lambda i: (i, 0)),
              _full((_H, _H)), _full((_H, _H)), _full((1, _H))],
    out_specs=pl.BlockSpec((_BE, _H), lambda i: (i, 0)),
    out_shape=jax.ShapeDtypeStruct((_EH, _H), jnp.float32),
    compiler_params=_PAR,
)


def _ff_body(g_ref, w2_ref, b2_ref, posr_ref, posc_ref, pf_ref, nf_ref):
    fm8 = (jnp.dot(_silu(g_ref[...]), w2_ref[...],
                   preferred_element_type=jnp.float32) + b2_ref[...])
    fm = fm8[:, 0:1]
    rd, d = _edge_attr(posr_ref[...], posc_ref[...])
    pf = fm * (rd / (d + 1e-8))
    pf_ref[...] = pf
    nf_ref[...] = -pf


_ff_call = pl.pallas_call(
    _ff_body,
    grid=(_GE,),
    in_specs=[pl.BlockSpec((_BE, _H), lambda i: (i, 0)),
              _full((_H, 8)), _full((1, 8)),
              pl.BlockSpec((_BE, 8), lambda i: (i, 0)),
              pl.BlockSpec((_BE, 8), lambda i: (i, 0))],
    out_specs=[pl.BlockSpec((_BE, 8), lambda i: (i, 0)),
               pl.BlockSpec((_BE, 8), lambda i: (i, 0))],
    out_shape=[jax.ShapeDtypeStruct((_EH, 8), jnp.float32),
               jax.ShapeDtypeStruct((_EH, 8), jnp.float32)],
    compiler_params=_PAR,
)


def _fin_body(a_ref, b_ref, c_ref, d_ref, out_ref):
    out_ref[...] = (a_ref[...] + b_ref[...] + c_ref[...] + d_ref[...])[:, :3]


_fin_call = pl.pallas_call(
    _fin_body,
    grid=(_GN,),
    in_specs=[pl.BlockSpec((_BN, 8), lambda i: (i, 0)),
              pl.BlockSpec((_BN, 8), lambda i: (i + _GN, 0)),
              pl.BlockSpec((_BN, 8), lambda i: (i, 0)),
              pl.BlockSpec((_BN, 8), lambda i: (i + _GN, 0))],
    out_specs=pl.BlockSpec((_BN, 3), lambda i: (i, 0)),
    out_shape=jax.ShapeDtypeStruct((_N, 3), jnp.float32),
    compiler_params=_PAR,
)


def kernel(pos, vel, masses, edge_index, ee_w1, ee_b1, ee_w2, ee_b2,
           msg_w1, msg_b1, msg_w2, msg_b2, fd_w1, fd_b1, fd_w2, fd_b2):
    f32 = jnp.float32
    row = edge_index[0]
    col = edge_index[1]
    pos8 = jnp.concatenate([pos, jnp.zeros((_N, 5), f32)], axis=1)
    eew1p = jnp.concatenate([ee_w1, jnp.zeros((4, _H), f32)], axis=0)
    fd_w2p = jnp.concatenate([fd_w2, jnp.zeros((_H, 7), f32)], axis=1)
    fd_b2p = jnp.concatenate([fd_b2, jnp.zeros((7,), f32)]).reshape(1, 8)
    w1t = msg_w1[:, :_H, :]
    w1b = msg_w1[:, _H:, :]
    zeros128 = jnp.zeros((_N // _NS, _H), f32)
    zeros8 = jnp.zeros((1250, 8), f32)

    g0 = _g0_call(vel, w1b[0], msg_b1[0].reshape(1, _H))
    posr = [_gather8_h[h](pos8, row) for h in range(2)]
    posc = [_gather8_h[h](pos8, col) for h in range(2)]
    gth = [_gather128_h[h](g0, row) for h in range(2)]
    emb, msg, p = [None, None], [None, None], [None, None]
    for h in range(2):
        emb[h], msg[h] = _am0_call(
            posr[h], posc[h], gth[h], eew1p, ee_b1.reshape(1, _H),
            ee_w2, ee_b2.reshape(1, _H), w1t[0], msg_w2[0],
            msg_b2[0].reshape(1, _H))
        p[h] = _scatter128_h[h](msg[h], col, zeros128)
    for l in range(1, _L):
        g = _g_call(p[0], p[0], p[1], p[1], w1b[l], msg_b1[l].reshape(1, _H))
        for h in range(2):
            gth[h] = _gather128_h[h](g, row)
        for h in range(2):
            msg[h] = _msg_call(emb[h], gth[h], w1t[l], msg_w2[l],
                               msg_b2[l].reshape(1, _H))
            p[h] = _scatter128_h[h](msg[h], col, zeros128)
    gf = _g_call(p[0], p[0], p[1], p[1], fd_w1, fd_b1.reshape(1, _H))
    q = [None, None]
    for h in range(2):
        gfr = _gather128_h[h](gf, row)
        pf, nf = _ff_call(gfr, fd_w2p, fd_b2p, posr[h], posc[h])
        q[h] = _scatter2_h[h](pf, nf, row, col, zeros8)
    return _fin_call(q[0], q[0], q[1], q[1])
